# baseline (speedup 1.0000x reference)
import functools

import jax
import jax.numpy as jnp
from jax import lax
from jax.experimental import pallas as pl
from jax.experimental.pallas import tpu as pltpu
from jax.experimental.pallas import tpu_sc as plsc

_BLK = 4096


def _vq_ids_kernel(x_ref, cb_ref, ids_ref, cbs_ref, c2_ref, cba_ref):
    @pl.when(pl.program_id(0) == 0)
    def _init():
        cbi = cb_ref[...]
        kk = cbi.shape[0]
        cbs_ref[...] = -2.0 * cbi
        c2_ref[...] = jnp.sum(cbi * cbi, axis=1, keepdims=True)
        iota = jax.lax.broadcasted_iota(jnp.int32, (kk, 1), 0)
        cba_ref[:, 0:1] = ((iota // 256) * 256).astype(jnp.float32)
        cba_ref[:, 1:2] = (iota % 256).astype(jnp.float32)
        cba_ref[:, 2:3] = jnp.ones((kk, 1), jnp.float32)

    x = x_ref[...]
    k = cbs_ref.shape[0]
    scoresT = jax.lax.dot_general(
        cbs_ref[...], x, (((1,), (1,)), ((), ())),
        preferred_element_type=jnp.float32,
    )
    distT = scoresT + c2_ref[...]
    mind = jnp.min(distT, axis=0)
    onehotT = (distT <= mind[None, :]).astype(jnp.float32)
    ids_aug = jax.lax.dot_general(
        onehotT, cba_ref[...], (((0,), (0,)), ((), ())),
        preferred_element_type=jnp.float32,
    )
    ids_ref[...] = (
        ids_aug[:, 0:1] + ids_aug[:, 1:2]).astype(jnp.int32).reshape(_BLK)
    count = ids_aug[:, 2:3]

    @pl.when(jnp.max(count) > 1.5)
    def _fix_ties():
        code_iota = jax.lax.broadcasted_iota(jnp.int32, distT.shape, 0)
        ids_t = jnp.min(
            jnp.where(distT <= mind[None, :], code_iota, k), axis=0
        ).astype(jnp.int32)
        ids_ref[...] = ids_t


def _tc_ids(x, codebook):
    n, d = x.shape
    k = codebook.shape[0]
    return pl.pallas_call(
        _vq_ids_kernel,
        grid=(n // _BLK,),
        in_specs=[
            pl.BlockSpec((_BLK, d), lambda i: (i, 0)),
            pl.BlockSpec((k, d), lambda i: (0, 0)),
        ],
        out_specs=pl.BlockSpec((_BLK,), lambda i: (i,)),
        out_shape=jax.ShapeDtypeStruct((n,), jnp.int32),
        scratch_shapes=[
            pltpu.VMEM((k, d), jnp.float32),
            pltpu.VMEM((k, 1), jnp.float32),
            pltpu.VMEM((k, 3), jnp.float32),
        ],
    )(x, codebook)


def _make_sc_gather(n, d):
    info = plsc.get_sparse_core_info()
    nw = info.num_cores * info.num_subcores
    b_per_w = n // nw
    chunk = 128
    n_chunks = b_per_w // chunk
    mesh = plsc.VectorSubcoreMesh(core_axis_name="c", subcore_axis_name="s")

    @functools.partial(
        pl.kernel, mesh=mesh,
        out_type=jax.ShapeDtypeStruct((n, 128), jnp.float32),
        scratch_types=[
            pltpu.VMEM((chunk,), jnp.int32),
            pltpu.VMEM((chunk, 128), jnp.float32),
            pltpu.SemaphoreType.DMA,
        ],
    )
    def sc_gather(table_hbm, idx_hbm, out_hbm, idx_v, rows_v, sem):
        wid = lax.axis_index("s") * info.num_cores + lax.axis_index("c")
        base = wid * b_per_w
        for i in range(n_chunks):
            off = base + i * chunk
            pltpu.sync_copy(idx_hbm.at[pl.ds(off, chunk)], idx_v)
            pltpu.async_copy(table_hbm.at[idx_v], rows_v, sem).wait()
            pltpu.sync_copy(rows_v, out_hbm.at[pl.ds(off, chunk)])

    return sc_gather


@functools.partial(jax.jit, static_argnames=())
def kernel(x, codebook):
    n, d = x.shape
    k = codebook.shape[0]
    cb_pad = jnp.concatenate(
        [codebook, jnp.zeros((k, 128 - d), jnp.float32)], axis=1)
    ids = _tc_ids(x, codebook)
    ek128 = _make_sc_gather(n, d)(cb_pad, ids)
    return (ek128[:, :d], ids)


# speedup vs baseline: 1.5992x; 1.5992x over previous
import functools

import jax
import jax.numpy as jnp
from jax.experimental import pallas as pl
from jax.experimental.pallas import tpu as pltpu

_BLK = 4096


def _vq_block_kernel(x_ref, cb_ref, ek_ref, ids_ref, cbs_ref, c2_ref, cba_ref):
    @pl.when(pl.program_id(0) == 0)
    def _init():
        cbi = cb_ref[...]
        kk = cbi.shape[0]
        cbs_ref[...] = -2.0 * cbi
        c2_ref[...] = jnp.sum(cbi * cbi, axis=1, keepdims=True)
        iota = jax.lax.broadcasted_iota(jnp.int32, (kk, 1), 0)
        cba_ref[:, :64] = cbi
        cba_ref[:, 64:65] = ((iota // 256) * 256).astype(jnp.float32)
        cba_ref[:, 65:66] = (iota % 256).astype(jnp.float32)
        cba_ref[:, 66:67] = jnp.ones((kk, 1), jnp.float32)

    x = x_ref[...]
    k = cbs_ref.shape[0]
    scoresT = jax.lax.dot_general(
        cbs_ref[...], x, (((1,), (1,)), ((), ())),
        preferred_element_type=jnp.float32,
    )
    distT = scoresT + c2_ref[...]
    mind = jnp.min(distT, axis=0)
    onehotT = (distT <= mind[None, :]).astype(jnp.float32)
    ek_aug = jax.lax.dot_general(
        onehotT, cba_ref[...], (((0,), (0,)), ((), ())),
        preferred_element_type=jnp.float32,
    )
    ek_ref[...] = ek_aug[:, :64]
    ids_ref[...] = (ek_aug[:, 64:65] + ek_aug[:, 65:66]).astype(jnp.int32)
    count = ek_aug[:, 66:67]

    @pl.when(jnp.max(count) > 1.5)
    def _fix_ties():
        code_iota = jax.lax.broadcasted_iota(jnp.int32, distT.shape, 0)
        ids_t = jnp.min(
            jnp.where(distT <= mind[None, :], code_iota, k), axis=0
        ).astype(jnp.int32)
        oh = (code_iota == ids_t[None, :]).astype(jnp.float32)
        ek_ref[...] = jax.lax.dot_general(
            oh, cba_ref[:, :64], (((0,), (0,)), ((), ())),
            preferred_element_type=jnp.float32,
        )
        ids_ref[...] = ids_t[:, None]


@functools.partial(jax.jit, static_argnames=())
def kernel(x, codebook):
    n, d = x.shape
    k = codebook.shape[0]
    grid = (n // _BLK,)
    ek, ids = pl.pallas_call(
        _vq_block_kernel,
        grid=grid,
        in_specs=[
            pl.BlockSpec((_BLK, d), lambda i: (i, 0)),
            pl.BlockSpec((k, d), lambda i: (0, 0)),
        ],
        out_specs=[
            pl.BlockSpec((_BLK, d), lambda i: (i, 0)),
            pl.BlockSpec((_BLK, 1), lambda i: (i, 0)),
        ],
        out_shape=[
            jax.ShapeDtypeStruct((n, d), jnp.float32),
            jax.ShapeDtypeStruct((n, 1), jnp.int32),
        ],
        scratch_shapes=[
            pltpu.VMEM((k, d), jnp.float32),
            pltpu.VMEM((k, 1), jnp.float32),
            pltpu.VMEM((k, 67), jnp.float32),
        ],
    )(x, codebook)
    return (ek, ids.reshape(n))
